# baseline (speedup 1.0000x reference)
import jax
import jax.numpy as jnp
from jax.experimental import pallas as pl
from jax.experimental.pallas import tpu as pltpu

E = 8
K = 2
D = 1024
F = 4096
T = 2048
S = T * K
BM = 256
P = S + E * BM
NB = P // BM


BF = 2048
NF = F // BF


def _ffn_body(be_ref, ba_ref, sel_ref, x_ref, w1_ref, w2_ref, y_ref, loads_ref):
    b = pl.program_id(0)
    f = pl.program_id(1)

    @pl.when((b == 0) & (f == 0))
    def _():
        sel = sel_ref[...]
        ee = jax.lax.broadcasted_iota(jnp.int32, (E, 32, 128), 0)
        cnt = jnp.sum((sel[None] == ee).astype(jnp.float32), axis=(1, 2))
        loads_ref[...] = (cnt / T)[None]

    @pl.when(ba_ref[b] > 0)
    def _():
        x = x_ref[...]
        h = jax.lax.dot_general(x, w1_ref[0], (((1,), (1,)), ((), ())),
                                preferred_element_type=jnp.float32)
        h = 0.5 * h * (1.0 + jax.lax.erf(h * 0.7071067811865476))
        yp = jax.lax.dot_general(h.astype(jnp.bfloat16), w2_ref[0],
                                 (((1,), (1,)), ((), ())),
                                 preferred_element_type=jnp.float32)

        @pl.when(f == 0)
        def _():
            y_ref[...] = yp

        @pl.when(f > 0)
        def _():
            y_ref[...] += yp


def _grouped_ffn(block_expert, block_active, sel2d, x_padded, W1, W2):
    grid_spec = pltpu.PrefetchScalarGridSpec(
        num_scalar_prefetch=2,
        grid=(NB, NF),
        in_specs=[
            pl.BlockSpec((32, 128), lambda b, f, be, ba: (0, 0)),
            pl.BlockSpec((BM, D), lambda b, f, be, ba: (b, 0)),
            pl.BlockSpec((1, BF, D), lambda b, f, be, ba: (be[b], f, 0)),
            pl.BlockSpec((1, D, BF), lambda b, f, be, ba: (be[b], 0, f)),
        ],
        out_specs=[
            pl.BlockSpec((BM, D), lambda b, f, be, ba: (b, 0)),
            pl.BlockSpec((1, E), lambda b, f, be, ba: (0, 0)),
        ],
    )
    return pl.pallas_call(
        _ffn_body,
        grid_spec=grid_spec,
        out_shape=[
            jax.ShapeDtypeStruct((P, D), jnp.float32),
            jax.ShapeDtypeStruct((1, E), jnp.float32),
        ],
        compiler_params=pltpu.CompilerParams(
            dimension_semantics=("arbitrary", "arbitrary"),
            vmem_limit_bytes=60 * 1024 * 1024,
        ),
    )(block_expert, block_active, sel2d, x_padded, W1, W2)


def kernel(hidden_states, selected_experts, expert_masks, W1, W2):
    sel = selected_experts.astype(jnp.int32)
    sel_flat = sel.reshape(-1)

    oh = (sel_flat[:, None] == jnp.arange(E, dtype=jnp.int32)[None, :])
    counts = jnp.sum(oh, axis=0, dtype=jnp.int32)
    rank = (jnp.cumsum(oh, axis=0, dtype=jnp.int32) - 1)
    rank = jnp.take_along_axis(rank, sel_flat[:, None], axis=1)[:, 0]
    pc = (counts + BM - 1) // BM
    cb = jnp.cumsum(pc)
    bstart = (cb - pc) * BM
    slot_pos = bstart[sel_flat] + rank
    tok_of_slot = (jnp.arange(S, dtype=jnp.int32) // K)
    gather_tok = jnp.zeros((P,), jnp.int32).at[slot_pos].set(tok_of_slot)
    bids = jnp.arange(NB, dtype=jnp.int32)
    block_expert = jnp.minimum(
        jnp.searchsorted(cb, bids, side="right"), E - 1).astype(jnp.int32)
    block_active = (bids < cb[E - 1]).astype(jnp.int32)

    x_padded = hidden_states[gather_tok].astype(jnp.bfloat16)
    y_padded, loads2d = _grouped_ffn(
        block_expert, block_active, sel.reshape(32, 128), x_padded,
        W1.astype(jnp.bfloat16), W2.astype(jnp.bfloat16))
    expert_outputs = y_padded[slot_pos].reshape(T, K, D)
    return expert_outputs, loads2d[0]


# speedup vs baseline: 1.0315x; 1.0315x over previous
import jax
import jax.numpy as jnp
from jax.experimental import pallas as pl
from jax.experimental.pallas import tpu as pltpu

E = 8
K = 2
D = 1024
F = 4096
T = 2048
S = T * K
BM = 1024
P = S + E * BM
NB = P // BM


BF = 1024
NF = F // BF


def _ffn_body(be_ref, ba_ref, sel_ref, x_ref, w1_ref, w2_ref, y_ref, loads_ref):
    b = pl.program_id(0)
    f = pl.program_id(1)

    @pl.when((b == 0) & (f == 0))
    def _():
        sel = sel_ref[...]
        ee = jax.lax.broadcasted_iota(jnp.int32, (E, 32, 128), 0)
        cnt = jnp.sum((sel[None] == ee).astype(jnp.float32), axis=(1, 2))
        loads_ref[...] = (cnt / T)[None]

    @pl.when(ba_ref[b] > 0)
    def _():
        x = x_ref[...]
        h = jax.lax.dot_general(x, w1_ref[0], (((1,), (1,)), ((), ())),
                                preferred_element_type=jnp.float32)
        h = 0.5 * h * (1.0 + jax.lax.erf(h * 0.7071067811865476))
        yp = jax.lax.dot_general(h, w2_ref[0], (((1,), (1,)), ((), ())),
                                 preferred_element_type=jnp.float32)

        @pl.when(f == 0)
        def _():
            y_ref[...] = yp

        @pl.when(f > 0)
        def _():
            y_ref[...] += yp


def _grouped_ffn(block_expert, block_active, sel2d, x_padded, W1, W2):
    grid_spec = pltpu.PrefetchScalarGridSpec(
        num_scalar_prefetch=2,
        grid=(NB, NF),
        in_specs=[
            pl.BlockSpec((32, 128), lambda b, f, be, ba: (0, 0)),
            pl.BlockSpec((BM, D), lambda b, f, be, ba: (b, 0)),
            pl.BlockSpec((1, BF, D), lambda b, f, be, ba: (be[b], f, 0)),
            pl.BlockSpec((1, D, BF), lambda b, f, be, ba: (be[b], 0, f)),
        ],
        out_specs=[
            pl.BlockSpec((BM, D), lambda b, f, be, ba: (b, 0)),
            pl.BlockSpec((1, E), lambda b, f, be, ba: (0, 0)),
        ],
    )
    return pl.pallas_call(
        _ffn_body,
        grid_spec=grid_spec,
        out_shape=[
            jax.ShapeDtypeStruct((P, D), jnp.float32),
            jax.ShapeDtypeStruct((1, E), jnp.float32),
        ],
        compiler_params=pltpu.CompilerParams(
            dimension_semantics=("arbitrary", "arbitrary"),
            vmem_limit_bytes=60 * 1024 * 1024,
        ),
    )(block_expert, block_active, sel2d, x_padded, W1, W2)


def kernel(hidden_states, selected_experts, expert_masks, W1, W2):
    sel = selected_experts.astype(jnp.int32)
    sel_flat = sel.reshape(-1)

    oh = (sel_flat[:, None] == jnp.arange(E, dtype=jnp.int32)[None, :])
    counts = jnp.sum(oh, axis=0, dtype=jnp.int32)
    rank = (jnp.cumsum(oh, axis=0, dtype=jnp.int32) - 1)
    rank = jnp.take_along_axis(rank, sel_flat[:, None], axis=1)[:, 0]
    pc = (counts + BM - 1) // BM
    cb = jnp.cumsum(pc)
    bstart = (cb - pc) * BM
    slot_pos = bstart[sel_flat] + rank
    tok_of_slot = (jnp.arange(S, dtype=jnp.int32) // K)
    gather_tok = jnp.zeros((P,), jnp.int32).at[slot_pos].set(tok_of_slot)
    bids = jnp.arange(NB, dtype=jnp.int32)
    block_expert = jnp.minimum(
        jnp.searchsorted(cb, bids, side="right"), E - 1).astype(jnp.int32)
    block_active = (bids < cb[E - 1]).astype(jnp.int32)

    x_padded = hidden_states[gather_tok]
    y_padded, loads2d = _grouped_ffn(
        block_expert, block_active, sel.reshape(32, 128), x_padded, W1, W2)
    expert_outputs = y_padded[slot_pos].reshape(T, K, D)
    return expert_outputs, loads2d[0]


# speedup vs baseline: 1.1070x; 1.0733x over previous
import jax
import jax.numpy as jnp
from jax.experimental import pallas as pl
from jax.experimental.pallas import tpu as pltpu

E = 8
K = 2
D = 1024
F = 4096
T = 2048
S = T * K
BM = 256
P = S + E * BM
NB = P // BM


BF = 2048
NF = F // BF


def _ffn_body(be_ref, ba_ref, sel_ref, x_ref, w1_ref, w2_ref, y_ref, loads_ref):
    b = pl.program_id(0)
    f = pl.program_id(1)

    @pl.when((b == 0) & (f == 0))
    def _():
        sel = sel_ref[...]
        ee = jax.lax.broadcasted_iota(jnp.int32, (E, 32, 128), 0)
        cnt = jnp.sum((sel[None] == ee).astype(jnp.float32), axis=(1, 2))
        loads_ref[...] = (cnt / T)[None]

    @pl.when(ba_ref[b] > 0)
    def _():
        x = x_ref[...]
        h = jax.lax.dot_general(x, w1_ref[0], (((1,), (1,)), ((), ())),
                                preferred_element_type=jnp.float32)
        h = 0.5 * h * (1.0 + jax.lax.erf(h * 0.7071067811865476))
        yp = jax.lax.dot_general(h, w2_ref[0], (((1,), (1,)), ((), ())),
                                 preferred_element_type=jnp.float32)

        @pl.when(f == 0)
        def _():
            y_ref[...] = yp

        @pl.when(f > 0)
        def _():
            y_ref[...] += yp


def _grouped_ffn(block_expert, block_active, sel2d, x_padded, W1, W2):
    grid_spec = pltpu.PrefetchScalarGridSpec(
        num_scalar_prefetch=2,
        grid=(NB, NF),
        in_specs=[
            pl.BlockSpec((32, 128), lambda b, f, be, ba: (0, 0)),
            pl.BlockSpec((BM, D), lambda b, f, be, ba: (b, 0)),
            pl.BlockSpec((1, BF, D), lambda b, f, be, ba: (be[b], f, 0)),
            pl.BlockSpec((1, D, BF), lambda b, f, be, ba: (be[b], 0, f)),
        ],
        out_specs=[
            pl.BlockSpec((BM, D), lambda b, f, be, ba: (b, 0)),
            pl.BlockSpec((1, E), lambda b, f, be, ba: (0, 0)),
        ],
    )
    return pl.pallas_call(
        _ffn_body,
        grid_spec=grid_spec,
        out_shape=[
            jax.ShapeDtypeStruct((P, D), jnp.float32),
            jax.ShapeDtypeStruct((1, E), jnp.float32),
        ],
        compiler_params=pltpu.CompilerParams(
            dimension_semantics=("arbitrary", "arbitrary"),
            vmem_limit_bytes=60 * 1024 * 1024,
        ),
    )(block_expert, block_active, sel2d, x_padded, W1, W2)


def kernel(hidden_states, selected_experts, expert_masks, W1, W2):
    sel = selected_experts.astype(jnp.int32)
    sel_flat = sel.reshape(-1)

    oh = (jnp.arange(E, dtype=jnp.int32)[:, None] == sel_flat[None, :])
    ohi = oh.astype(jnp.int32)
    counts = jnp.sum(ohi, axis=1)
    rank_all = jnp.cumsum(ohi, axis=1) - 1
    pc = (counts + BM - 1) // BM
    cb = jnp.cumsum(pc)
    bstart = (cb - pc) * BM
    slot_pos = jnp.sum(jnp.where(oh, rank_all + bstart[:, None], 0),
                       axis=0).astype(jnp.int32)
    tok_of_slot = (jnp.arange(S, dtype=jnp.int32) // K)
    gather_tok = jnp.zeros((P,), jnp.int32).at[slot_pos].set(tok_of_slot)
    bids = jnp.arange(NB, dtype=jnp.int32)
    block_expert = jnp.minimum(
        jnp.searchsorted(cb, bids, side="right"), E - 1).astype(jnp.int32)
    block_active = (bids < cb[E - 1]).astype(jnp.int32)

    x_padded = hidden_states[gather_tok]
    y_padded, loads2d = _grouped_ffn(
        block_expert, block_active, sel.reshape(32, 128), x_padded, W1, W2)
    expert_outputs = y_padded[slot_pos].reshape(T, K, D)
    return expert_outputs, loads2d[0]
